# Initial kernel scaffold; baseline (speedup 1.0000x reference)
import jax
import jax.numpy as jnp
from jax.experimental import pallas as pl


def kernel(preq_latents, codebook):
    raise NotImplementedError("write your pallas kernel here")


# speedup vs baseline: 2.1739x; 2.1739x over previous
import functools

import jax
import jax.numpy as jnp
from jax import lax
from jax.experimental import pallas as pl
from jax.experimental.pallas import tpu as pltpu
from jax.experimental.pallas import tpu_sc as plsc

N = 18432
D = 64
K = 1024
BN = 2048
NB = N // BN

NC = 2
NS = 16
NW = NC * NS
RPW = N // NW
GCH = 64
NCH = RPW // GCH


def _tc_body(lat_ref, cbn_ref, cbsq_ref, dist_ref, idx_ref, loss_ref):
    cbn = cbn_ref[...]
    lat = lat_ref[...]
    lat_sq = jnp.sum(lat * lat, axis=1, keepdims=True)
    mm = lax.dot_general(
        lat, cbn,
        (((1,), (1,)), ((), ())),
        preferred_element_type=jnp.float32)
    dist = lat_sq - 2.0 * mm + cbsq_ref[...]
    dist_ref[...] = dist
    min_d = jnp.min(dist, axis=1, keepdims=True)
    iota_k = lax.broadcasted_iota(jnp.int32, (BN, K), 1)
    idx = jnp.min(jnp.where(dist == min_d, iota_k, K), axis=1)
    idx_ref[0, 0, :] = idx

    i = pl.program_id(0)

    @pl.when(i == 0)
    def _():
        loss_ref[0, 0] = 0.0

    loss_ref[0, 0] += jnp.sum(min_d)


_tc_call = pl.pallas_call(
    _tc_body,
    grid=(NB,),
    in_specs=[
        pl.BlockSpec((BN, D), lambda i: (i, 0)),
        pl.BlockSpec((K, D), lambda i: (0, 0)),
        pl.BlockSpec((1, K), lambda i: (0, 0)),
    ],
    out_specs=[
        pl.BlockSpec((BN, K), lambda i: (i, 0)),
        pl.BlockSpec((1, 1, BN), lambda i: (i, 0, 0)),
        pl.BlockSpec(memory_space=pltpu.SMEM),
    ],
    out_shape=[
        jax.ShapeDtypeStruct((N, K), jnp.float32),
        jax.ShapeDtypeStruct((NB, 1, BN), jnp.int32),
        jax.ShapeDtypeStruct((1, 1), jnp.float32),
    ],
)


@functools.lru_cache(maxsize=1)
def _make_sc_gather():
    @functools.partial(
        pl.kernel,
        mesh=plsc.VectorSubcoreMesh(core_axis_name="c", subcore_axis_name="s"),
        out_type=jax.ShapeDtypeStruct((N, D), jnp.float32),
        scratch_types=[
            pltpu.VMEM((NCH, GCH), jnp.int32),
            pltpu.VMEM((RPW, D), jnp.float32),
            pltpu.SemaphoreType.DMA,
        ],
        compiler_params=pltpu.CompilerParams(use_tc_tiling_on_sc=False),
    )
    def _sc_gather(cbn_hbm, idx_hbm, out_hbm, idx_v, rows_v, sem):
        wid = lax.axis_index("s") * NC + lax.axis_index("c")
        base = wid * RPW
        pltpu.sync_copy(idx_hbm.at[wid], idx_v)
        handles = [
            pltpu.async_copy(cbn_hbm.at[idx_v.at[j]],
                             rows_v.at[pl.ds(j * GCH, GCH)], sem)
            for j in range(NCH)
        ]
        for h in handles:
            h.wait()
        pltpu.sync_copy(rows_v, out_hbm.at[pl.ds(base, RPW)])

    return _sc_gather


def kernel(preq_latents, codebook):
    B, C, H, W = preq_latents.shape
    lat = jnp.transpose(preq_latents, (0, 2, 3, 1)).reshape(N, D)
    norm = jnp.linalg.norm(codebook, axis=1, keepdims=True)
    cbn = codebook / jnp.maximum(norm, 1e-12)
    cb_sq = jnp.sum(cbn ** 2, axis=1)[None, :]
    distances, idx3, loss_sum = _tc_call(lat, cbn, cb_sq)
    indices = idx3.reshape(N)
    q = _make_sc_gather()(cbn, indices.reshape(NW, NCH, GCH))
    st = jnp.transpose(q.reshape(B, H, W, C), (0, 3, 1, 2))
    loss = loss_sum[0, 0] / jnp.float32(N * D)
    return (st, preq_latents, loss, loss, indices, distances)


# speedup vs baseline: 2.4528x; 1.1283x over previous
import functools

import jax
import jax.numpy as jnp
from jax import lax
from jax.experimental import pallas as pl
from jax.experimental.pallas import tpu as pltpu
from jax.experimental.pallas import tpu_sc as plsc

N = 18432
D = 64
K = 1024
BN = 2048
NB = N // BN

NC = 2
NS = 16
NW = NC * NS
RPW = N // NW
GCH = 64
NCH = RPW // GCH


def _tc_body(lat_ref, cbn_ref, cbsq_ref, dist_ref, idx_ref, loss_ref):
    cbn = cbn_ref[...]
    lat = lat_ref[...]
    lat_sq = jnp.sum(lat * lat, axis=1, keepdims=True)
    mm = lax.dot_general(
        lat, cbn,
        (((1,), (1,)), ((), ())),
        preferred_element_type=jnp.float32)
    dist = lat_sq - 2.0 * mm + cbsq_ref[...]
    dist_ref[...] = dist
    m = dist[:, 0:128]
    bj = jnp.zeros((BN, 128), jnp.int32)
    for j in range(1, K // 128):
        dj = dist[:, 128 * j:128 * (j + 1)]
        lt = dj < m
        m = jnp.minimum(m, dj)
        bj = jnp.where(lt, jnp.int32(j), bj)
    min_d = jnp.min(m, axis=1, keepdims=True)
    k_cand = bj * 128 + lax.broadcasted_iota(jnp.int32, (BN, 128), 1)
    idx = jnp.min(jnp.where(m == min_d, k_cand, K), axis=1, keepdims=True)
    idx_ref[...] = idx

    i = pl.program_id(0)

    @pl.when(i == 0)
    def _():
        loss_ref[0, 0] = 0.0

    loss_ref[0, 0] += jnp.sum(min_d)


_tc_call = pl.pallas_call(
    _tc_body,
    grid=(NB,),
    in_specs=[
        pl.BlockSpec((BN, D), lambda i: (i, 0)),
        pl.BlockSpec((K, D), lambda i: (0, 0)),
        pl.BlockSpec((1, K), lambda i: (0, 0)),
    ],
    out_specs=[
        pl.BlockSpec((BN, K), lambda i: (i, 0)),
        pl.BlockSpec((BN, 1), lambda i: (i, 0)),
        pl.BlockSpec(memory_space=pltpu.SMEM),
    ],
    out_shape=[
        jax.ShapeDtypeStruct((N, K), jnp.float32),
        jax.ShapeDtypeStruct((N, 1), jnp.int32),
        jax.ShapeDtypeStruct((1, 1), jnp.float32),
    ],
)


@functools.lru_cache(maxsize=1)
def _make_sc_gather():
    @functools.partial(
        pl.kernel,
        mesh=plsc.VectorSubcoreMesh(core_axis_name="c", subcore_axis_name="s"),
        out_type=jax.ShapeDtypeStruct((N, D), jnp.float32),
        scratch_types=[
            pltpu.VMEM((NCH, GCH), jnp.int32),
            pltpu.VMEM((RPW, D), jnp.float32),
            pltpu.SemaphoreType.DMA,
        ],
        compiler_params=pltpu.CompilerParams(use_tc_tiling_on_sc=False),
    )
    def _sc_gather(cbn_hbm, idx_hbm, out_hbm, idx_v, rows_v, sem):
        wid = lax.axis_index("s") * NC + lax.axis_index("c")
        base = wid * RPW
        pltpu.sync_copy(idx_hbm.at[wid], idx_v)
        handles = [
            pltpu.async_copy(cbn_hbm.at[idx_v.at[j]],
                             rows_v.at[pl.ds(j * GCH, GCH)], sem)
            for j in range(NCH)
        ]
        for h in handles:
            h.wait()
        pltpu.sync_copy(rows_v, out_hbm.at[pl.ds(base, RPW)])

    return _sc_gather


def kernel(preq_latents, codebook):
    B, C, H, W = preq_latents.shape
    lat = jnp.transpose(preq_latents, (0, 2, 3, 1)).reshape(N, D)
    norm = jnp.linalg.norm(codebook, axis=1, keepdims=True)
    cbn = codebook / jnp.maximum(norm, 1e-12)
    cb_sq = jnp.sum(cbn ** 2, axis=1)[None, :]
    distances, idx2, loss_sum = _tc_call(lat, cbn, cb_sq)
    indices = idx2.reshape(N)
    q = _make_sc_gather()(cbn, indices.reshape(NW, NCH, GCH))
    st = jnp.transpose(q.reshape(B, H, W, C), (0, 3, 1, 2))
    loss = loss_sum[0, 0] / jnp.float32(N * D)
    return (st, preq_latents, loss, loss, indices, distances)
